# Initial kernel scaffold; baseline (speedup 1.0000x reference)
import jax
import jax.numpy as jnp
from jax.experimental import pallas as pl


def kernel(h, vn_h, segment_ids, W, b):
    raise NotImplementedError("write your pallas kernel here")


# speedup vs baseline: 1.2017x; 1.2017x over previous
import functools

import jax
import jax.numpy as jnp
from jax import lax
from jax.experimental import pallas as pl
from jax.experimental.pallas import tpu as pltpu
from jax.experimental.pallas import tpu_sc as plsc

N = 100000
M = 512
D = 128
CHUNK = 80
NCHUNKS = N // CHUNK
NC = 2
NS = 16
NW = NC * NS
NV = D // 16

_mesh = plsc.VectorSubcoreMesh(core_axis_name="c", subcore_axis_name="s")


def _worker_id():
    return lax.axis_index("s") * NC + lax.axis_index("c")


def _chunk_range(wid):
    base, rem = NCHUNKS // NW, NCHUNKS % NW
    start = wid * base + jnp.minimum(wid, rem)
    cnt = base + jnp.where(wid < rem, 1, 0)
    return start.astype(jnp.int32), cnt.astype(jnp.int32)


@functools.partial(
    pl.kernel,
    mesh=_mesh,
    out_type=jax.ShapeDtypeStruct((NW, M, D), jnp.float32),
    scratch_types=[
        pltpu.VMEM((CHUNK + 16,), jnp.int32),
        pltpu.VMEM((CHUNK, D), jnp.float32),
        pltpu.VMEM((M, D), jnp.float32),
    ],
)
def _segsum(h_hbm, seg_hbm, out_hbm, idx_v, rows_v, pool_v):
    wid = _worker_id()
    zero = jnp.zeros((16,), jnp.float32)

    def zrow(r, carry):
        for j in range(NV):
            pool_v[r, pl.ds(j * 16, 16)] = zero
        return carry

    lax.fori_loop(0, M, zrow, 0)
    start, cnt = _chunk_range(wid)

    def chunk_body(k, carry):
        c = start + k
        pltpu.sync_copy(seg_hbm.at[pl.ds(c * CHUNK, CHUNK)],
                        idx_v.at[pl.ds(0, CHUNK)])
        pltpu.sync_copy(h_hbm.at[pl.ds(c * CHUNK, CHUNK)], rows_v)

        def row_body(r, carry2):
            s = idx_v[pl.ds(r, 16)][0]
            for j in range(NV):
                sl = pl.ds(j * 16, 16)
                pool_v[s, sl] = pool_v[s, sl] + rows_v[r, sl]
            return carry2

        lax.fori_loop(0, CHUNK, row_body, 0)
        return carry

    lax.fori_loop(0, cnt, chunk_body, 0)
    pltpu.sync_copy(pool_v, out_hbm.at[wid])


def _fc_body(part_ref, vn_ref, w_ref, b_ref, out_ref):
    pool = jnp.sum(part_ref[...], axis=0)
    x = vn_ref[...] + pool
    y = lax.dot_general(x, w_ref[...], (((1,), (1,)), ((), ())),
                        preferred_element_type=jnp.float32)
    out_ref[...] = vn_ref[...] + jnp.maximum(y + b_ref[...][None, :], 0.0)


def _fc(partials, vn_h, W, b):
    return pl.pallas_call(
        _fc_body,
        out_shape=jax.ShapeDtypeStruct((M, D), jnp.float32),
    )(partials, vn_h, W, b)


@functools.partial(
    pl.kernel,
    mesh=_mesh,
    out_type=jax.ShapeDtypeStruct((N, D), jnp.float32),
    scratch_types=[
        pltpu.VMEM((CHUNK + 16,), jnp.int32),
        pltpu.VMEM((CHUNK, D), jnp.float32),
        pltpu.VMEM((M, D), jnp.float32),
    ],
)
def _bcast(h_hbm, seg_hbm, vn_hbm, out_hbm, idx_v, hrow_v, vn_v):
    wid = _worker_id()
    pltpu.sync_copy(vn_hbm, vn_v)
    start, cnt = _chunk_range(wid)

    def chunk_body(k, carry):
        c = start + k
        pltpu.sync_copy(seg_hbm.at[pl.ds(c * CHUNK, CHUNK)],
                        idx_v.at[pl.ds(0, CHUNK)])
        pltpu.sync_copy(h_hbm.at[pl.ds(c * CHUNK, CHUNK)], hrow_v)

        def row_body(r, carry2):
            s = idx_v[pl.ds(r, 16)][0]
            for j in range(NV):
                sl = pl.ds(j * 16, 16)
                hrow_v[r, sl] = hrow_v[r, sl] + vn_v[s, sl]
            return carry2

        lax.fori_loop(0, CHUNK, row_body, 0)
        pltpu.sync_copy(hrow_v, out_hbm.at[pl.ds(c * CHUNK, CHUNK)])
        return carry

    lax.fori_loop(0, cnt, chunk_body, 0)


def kernel(h, vn_h, segment_ids, W, b):
    seg = segment_ids.astype(jnp.int32)
    partials = _segsum(h, seg)
    vn_new = _fc(partials, vn_h, W, b)
    h_new = _bcast(h, seg, vn_new)
    return (h_new, vn_new)


# speedup vs baseline: 2.6945x; 2.2422x over previous
import functools

import jax
import jax.numpy as jnp
from jax import lax
from jax.experimental import pallas as pl
from jax.experimental.pallas import tpu as pltpu
from jax.experimental.pallas import tpu_sc as plsc

N = 100000
M = 512
D = 128
CHUNK = 160
NCHUNKS = N // CHUNK
NC = 2
NS = 16
NW = NC * NS
NV = D // 16

_mesh = plsc.VectorSubcoreMesh(core_axis_name="c", subcore_axis_name="s")


def _worker_id():
    return lax.axis_index("s") * NC + lax.axis_index("c")


def _chunk_range(wid):
    base, rem = NCHUNKS // NW, NCHUNKS % NW
    start = wid * base + jnp.minimum(wid, rem)
    cnt = base + jnp.where(wid < rem, 1, 0)
    return start.astype(jnp.int32), cnt.astype(jnp.int32)


@functools.partial(
    pl.kernel,
    mesh=_mesh,
    out_type=jax.ShapeDtypeStruct((NW, M, D), jnp.float32),
    scratch_types=[
        pltpu.VMEM((CHUNK + 16,), jnp.int32),
        pltpu.VMEM((CHUNK, D), jnp.float32),
        pltpu.VMEM((M, D), jnp.float32),
    ],
)
def _segsum(h_hbm, seg_hbm, out_hbm, idx_v, rows_v, pool_v):
    wid = _worker_id()
    zero = jnp.zeros((16,), jnp.float32)

    def zrow(r, carry):
        for j in range(NV):
            pool_v[r, pl.ds(j * 16, 16)] = zero
        return carry

    lax.fori_loop(0, M, zrow, 0)
    start, cnt = _chunk_range(wid)

    def chunk_body(k, carry):
        c = start + k
        pltpu.sync_copy(seg_hbm.at[pl.ds(c * CHUNK, CHUNK)],
                        idx_v.at[pl.ds(0, CHUNK)])
        pltpu.sync_copy(h_hbm.at[pl.ds(c * CHUNK, CHUNK)], rows_v)

        def group_body(g, carry2):
            ids = idx_v[pl.ds(g * 16, 16)]
            mn = ids[0]
            uniform = mn == ids[15]

            @pl.when(uniform)
            def _fast():
                for j in range(NV):
                    sl = pl.ds(j * 16, 16)
                    acc = rows_v[g * 16, sl]
                    for r in range(1, 16):
                        acc = acc + rows_v[g * 16 + r, sl]
                    pool_v[mn, sl] = pool_v[mn, sl] + acc

            @pl.when(jnp.logical_not(uniform))
            def _slow():
                def row_body(r, carry3):
                    s = idx_v[pl.ds(r, 16)][0]
                    for j in range(NV):
                        sl = pl.ds(j * 16, 16)
                        pool_v[s, sl] = pool_v[s, sl] + rows_v[r, sl]
                    return carry3

                lax.fori_loop(g * 16, (g + 1) * 16, row_body, 0)

            return carry2

        lax.fori_loop(0, CHUNK // 16, group_body, 0)
        return carry

    lax.fori_loop(0, cnt, chunk_body, 0)
    pltpu.sync_copy(pool_v, out_hbm.at[wid])


def _fc_body(part_ref, vn_ref, w_ref, b_ref, out_ref):
    pool = jnp.sum(part_ref[...], axis=0)
    x = vn_ref[...] + pool
    y = lax.dot_general(x, w_ref[...], (((1,), (1,)), ((), ())),
                        preferred_element_type=jnp.float32)
    out_ref[...] = vn_ref[...] + jnp.maximum(y + b_ref[...][None, :], 0.0)


def _fc(partials, vn_h, W, b):
    return pl.pallas_call(
        _fc_body,
        out_shape=jax.ShapeDtypeStruct((M, D), jnp.float32),
    )(partials, vn_h, W, b)


@functools.partial(
    pl.kernel,
    mesh=_mesh,
    out_type=jax.ShapeDtypeStruct((N, D), jnp.float32),
    scratch_types=[
        pltpu.VMEM((CHUNK + 16,), jnp.int32),
        pltpu.VMEM((CHUNK, D), jnp.float32),
        pltpu.VMEM((M, D), jnp.float32),
    ],
)
def _bcast(h_hbm, seg_hbm, vn_hbm, out_hbm, idx_v, hrow_v, vn_v):
    wid = _worker_id()
    pltpu.sync_copy(vn_hbm, vn_v)
    start, cnt = _chunk_range(wid)

    def chunk_body(k, carry):
        c = start + k
        pltpu.sync_copy(seg_hbm.at[pl.ds(c * CHUNK, CHUNK)],
                        idx_v.at[pl.ds(0, CHUNK)])
        pltpu.sync_copy(h_hbm.at[pl.ds(c * CHUNK, CHUNK)], hrow_v)

        def group_body(g, carry2):
            ids = idx_v[pl.ds(g * 16, 16)]
            mn = ids[0]
            uniform = mn == ids[15]

            @pl.when(uniform)
            def _fast():
                for j in range(NV):
                    sl = pl.ds(j * 16, 16)
                    vnr = vn_v[mn, sl]
                    for r in range(16):
                        hrow_v[g * 16 + r, sl] = hrow_v[g * 16 + r, sl] + vnr

            @pl.when(jnp.logical_not(uniform))
            def _slow():
                def row_body(r, carry3):
                    s = idx_v[pl.ds(r, 16)][0]
                    for j in range(NV):
                        sl = pl.ds(j * 16, 16)
                        hrow_v[r, sl] = hrow_v[r, sl] + vn_v[s, sl]
                    return carry3

                lax.fori_loop(g * 16, (g + 1) * 16, row_body, 0)

            return carry2

        lax.fori_loop(0, CHUNK // 16, group_body, 0)
        pltpu.sync_copy(hrow_v, out_hbm.at[pl.ds(c * CHUNK, CHUNK)])
        return carry

    lax.fori_loop(0, cnt, chunk_body, 0)


def kernel(h, vn_h, segment_ids, W, b):
    seg = segment_ids.astype(jnp.int32)
    partials = _segsum(h, seg)
    vn_new = _fc(partials, vn_h, W, b)
    h_new = _bcast(h, seg, vn_new)
    return (h_new, vn_new)


# speedup vs baseline: 3.6018x; 1.3367x over previous
import functools

import jax
import jax.numpy as jnp
from jax import lax
from jax.experimental import pallas as pl
from jax.experimental.pallas import tpu as pltpu
from jax.experimental.pallas import tpu_sc as plsc

N = 100000
M = 512
D = 128
CHUNK = 160
NCHUNKS = N // CHUNK
BCHUNK = 80
NBCHUNKS = N // BCHUNK
NC = 2
NS = 16
NW = NC * NS
NV = D // 16

_mesh = plsc.VectorSubcoreMesh(core_axis_name="c", subcore_axis_name="s")


def _worker_id():
    return lax.axis_index("s") * NC + lax.axis_index("c")


def _chunk_range(wid, nchunks):
    base, rem = nchunks // NW, nchunks % NW
    start = wid * base + jnp.minimum(wid, rem)
    cnt = base + jnp.where(wid < rem, 1, 0)
    return start.astype(jnp.int32), cnt.astype(jnp.int32)


@functools.partial(
    pl.kernel,
    mesh=_mesh,
    out_type=jax.ShapeDtypeStruct((NW, M, D), jnp.float32),
    scratch_types=[
        pltpu.VMEM((CHUNK + 16,), jnp.int32),
        pltpu.VMEM((CHUNK, D), jnp.float32),
        pltpu.VMEM((CHUNK, D), jnp.float32),
        pltpu.VMEM((M, D), jnp.float32),
        pltpu.SemaphoreType.DMA,
        pltpu.SemaphoreType.DMA,
    ],
)
def _segsum(h_hbm, seg_hbm, out_hbm, idx_v, rows_a, rows_b, pool_v,
            sem_a, sem_b):
    wid = _worker_id()
    zero = jnp.zeros((16,), jnp.float32)
    bufs = ((rows_a, sem_a), (rows_b, sem_b))

    def _ld(c, rbuf, sem):
        pltpu.async_copy(h_hbm.at[pl.ds(c * CHUNK, CHUNK)], rbuf, sem)

    def _wt(rbuf, sem):
        pltpu.make_async_copy(h_hbm.at[pl.ds(0, CHUNK)], rbuf, sem).wait()

    def zrow(r, carry):
        for j in range(NV):
            pool_v[r, pl.ds(j * 16, 16)] = zero
        return carry

    lax.fori_loop(0, M, zrow, 0)
    start, cnt = _chunk_range(wid, NCHUNKS)
    end = start + cnt

    @pl.when(cnt > 0)
    def _p0():
        _ld(start, rows_a, sem_a)

    @pl.when(cnt > 1)
    def _p1():
        _ld(start + 1, rows_b, sem_b)

    def chunk_body(c, rows_v, sem):
        pltpu.sync_copy(seg_hbm.at[pl.ds(c * CHUNK, CHUNK)],
                        idx_v.at[pl.ds(0, CHUNK)])
        _wt(rows_v, sem)

        def group_body(g, carry2):
            ids = idx_v[pl.ds(g * 16, 16)]
            mn = ids[0]
            uniform = mn == ids[15]

            @pl.when(uniform)
            def _fast():
                for j in range(NV):
                    sl = pl.ds(j * 16, 16)
                    acc = rows_v[g * 16, sl]
                    for r in range(1, 16):
                        acc = acc + rows_v[g * 16 + r, sl]
                    pool_v[mn, sl] = pool_v[mn, sl] + acc

            @pl.when(jnp.logical_not(uniform))
            def _slow():
                def row_body(r, carry3):
                    s = idx_v[pl.ds(r, 16)][0]
                    for j in range(NV):
                        sl = pl.ds(j * 16, 16)
                        pool_v[s, sl] = pool_v[s, sl] + rows_v[r, sl]
                    return carry3

                lax.fori_loop(g * 16, (g + 1) * 16, row_body, 0)

            return carry2

        lax.fori_loop(0, CHUNK // 16, group_body, 0)

        @pl.when(c + 2 < end)
        def _next():
            _ld(c + 2, rows_v, sem)

    def pair_body(p, carry):
        base = start + p * 2
        for b, (rbuf, sem) in enumerate(bufs):
            c = base + b

            @pl.when(c < end)
            def _(c=c, rbuf=rbuf, sem=sem):
                chunk_body(c, rbuf, sem)

        return carry

    lax.fori_loop(0, (cnt + 1) // 2, pair_body, 0)
    pltpu.sync_copy(pool_v, out_hbm.at[wid])


def _fc_body(part_ref, vn_ref, w_ref, b_ref, out_ref):
    pool = jnp.sum(part_ref[...], axis=0)
    x = vn_ref[...] + pool
    y = lax.dot_general(x, w_ref[...], (((1,), (1,)), ((), ())),
                        preferred_element_type=jnp.float32)
    out_ref[...] = vn_ref[...] + jnp.maximum(y + b_ref[...][None, :], 0.0)


def _fc(partials, vn_h, W, b):
    return pl.pallas_call(
        _fc_body,
        out_shape=jax.ShapeDtypeStruct((M, D), jnp.float32),
    )(partials, vn_h, W, b)


@functools.partial(
    pl.kernel,
    mesh=_mesh,
    out_type=jax.ShapeDtypeStruct((N, D), jnp.float32),
    scratch_types=[
        pltpu.VMEM((BCHUNK + 16,), jnp.int32),
        pltpu.VMEM((BCHUNK, D), jnp.float32),
        pltpu.VMEM((BCHUNK, D), jnp.float32),
        pltpu.VMEM((BCHUNK, D), jnp.float32),
        pltpu.VMEM((BCHUNK, D), jnp.float32),
        pltpu.VMEM((M, D), jnp.float32),
        pltpu.SemaphoreType.DMA,
        pltpu.SemaphoreType.DMA,
        pltpu.SemaphoreType.DMA,
        pltpu.SemaphoreType.DMA,
    ],
)
def _bcast(h_hbm, seg_hbm, vn_hbm, out_hbm, idx_v, hin_a, hin_b,
           hout_a, hout_b, vn_v, semi_a, semi_b, semo_a, semo_b):
    wid = _worker_id()
    bufs = ((hin_a, hout_a, semi_a, semo_a), (hin_b, hout_b, semi_b, semo_b))

    def _ld(c, hin, semi):
        pltpu.async_copy(h_hbm.at[pl.ds(c * BCHUNK, BCHUNK)], hin, semi)

    def _wt_ld(hin, semi):
        pltpu.make_async_copy(h_hbm.at[pl.ds(0, BCHUNK)], hin, semi).wait()

    def _st(c, hout, semo):
        pltpu.async_copy(hout, out_hbm.at[pl.ds(c * BCHUNK, BCHUNK)], semo)

    def _wt_st(hout, semo):
        pltpu.make_async_copy(hout, out_hbm.at[pl.ds(0, BCHUNK)], semo).wait()

    pltpu.sync_copy(vn_hbm, vn_v)
    start, cnt = _chunk_range(wid, NBCHUNKS)
    end = start + cnt

    @pl.when(cnt > 0)
    def _p0():
        _ld(start, hin_a, semi_a)

    @pl.when(cnt > 1)
    def _p1():
        _ld(start + 1, hin_b, semi_b)

    def chunk_body(c, hin, hout, semi, semo):
        pltpu.sync_copy(seg_hbm.at[pl.ds(c * BCHUNK, BCHUNK)],
                        idx_v.at[pl.ds(0, BCHUNK)])
        _wt_ld(hin, semi)

        @pl.when(c - 2 >= start)
        def _drain():
            _wt_st(hout, semo)

        def group_body(g, carry2):
            ids = idx_v[pl.ds(g * 16, 16)]
            mn = ids[0]
            uniform = mn == ids[15]

            @pl.when(uniform)
            def _fast():
                for j in range(NV):
                    sl = pl.ds(j * 16, 16)
                    vnr = vn_v[mn, sl]
                    for r in range(16):
                        hout[g * 16 + r, sl] = hin[g * 16 + r, sl] + vnr

            @pl.when(jnp.logical_not(uniform))
            def _slow():
                def row_body(r, carry3):
                    s = idx_v[pl.ds(r, 16)][0]
                    for j in range(NV):
                        sl = pl.ds(j * 16, 16)
                        hout[r, sl] = hin[r, sl] + vn_v[s, sl]
                    return carry3

                lax.fori_loop(g * 16, (g + 1) * 16, row_body, 0)

            return carry2

        lax.fori_loop(0, BCHUNK // 16, group_body, 0)
        _st(c, hout, semo)

        @pl.when(c + 2 < end)
        def _next():
            _ld(c + 2, hin, semi)

    def pair_body(p, carry):
        base = start + p * 2
        for b, (hin, hout, semi, semo) in enumerate(bufs):
            c = base + b

            @pl.when(c < end)
            def _(c=c, hin=hin, hout=hout, semi=semi, semo=semo):
                chunk_body(c, hin, hout, semi, semo)

        return carry

    lax.fori_loop(0, (cnt + 1) // 2, pair_body, 0)
    for b, (hin, hout, semi, semo) in enumerate(bufs):
        @pl.when(cnt > b)
        def _(hout=hout, semo=semo):
            _wt_st(hout, semo)


def kernel(h, vn_h, segment_ids, W, b):
    seg = segment_ids.astype(jnp.int32)
    partials = _segsum(h, seg)
    vn_new = _fc(partials, vn_h, W, b)
    h_new = _bcast(h, seg, vn_new)
    return (h_new, vn_new)


# speedup vs baseline: 4.0558x; 1.1260x over previous
import functools

import jax
import jax.numpy as jnp
from jax import lax
from jax.experimental import pallas as pl
from jax.experimental.pallas import tpu as pltpu
from jax.experimental.pallas import tpu_sc as plsc

N = 100000
M = 512
D = 128
CHUNK = 160
NCHUNKS = N // CHUNK
BCHUNK = 80
NBCHUNKS = N // BCHUNK
NC = 2
NS = 16
NW = NC * NS
NV = D // 16

_mesh = plsc.VectorSubcoreMesh(core_axis_name="c", subcore_axis_name="s")


def _worker_id():
    return lax.axis_index("s") * NC + lax.axis_index("c")


def _chunk_range(wid, nchunks):
    base, rem = nchunks // NW, nchunks % NW
    start = wid * base + jnp.minimum(wid, rem)
    cnt = base + jnp.where(wid < rem, 1, 0)
    return start.astype(jnp.int32), cnt.astype(jnp.int32)


@functools.partial(
    pl.kernel,
    mesh=_mesh,
    out_type=jax.ShapeDtypeStruct((NW, M, D), jnp.float32),
    scratch_types=[
        pltpu.VMEM((3200 + 16,), jnp.int32),
        pltpu.VMEM((CHUNK, D), jnp.float32),
        pltpu.VMEM((CHUNK, D), jnp.float32),
        pltpu.VMEM((M, D), jnp.float32),
        pltpu.SemaphoreType.DMA,
        pltpu.SemaphoreType.DMA,
    ],
)
def _segsum(h_hbm, seg_hbm, out_hbm, idx_v, rows_a, rows_b, pool_v,
            sem_a, sem_b):
    wid = _worker_id()
    zero = jnp.zeros((16,), jnp.float32)
    bufs = ((rows_a, sem_a), (rows_b, sem_b))

    def _ld(c, rbuf, sem):
        pltpu.async_copy(h_hbm.at[pl.ds(c * CHUNK, CHUNK)], rbuf, sem)

    def _wt(rbuf, sem):
        pltpu.make_async_copy(h_hbm.at[pl.ds(0, CHUNK)], rbuf, sem).wait()

    def zrow(r, carry):
        for j in range(NV):
            pool_v[r, pl.ds(j * 16, 16)] = zero
        return carry

    lax.fori_loop(0, M, zrow, 0)
    start, cnt = _chunk_range(wid, NCHUNKS)
    end = start + cnt

    @pl.when(cnt > 0)
    def _p0():
        _ld(start, rows_a, sem_a)

    @pl.when(cnt > 1)
    def _p1():
        _ld(start + 1, rows_b, sem_b)

    off = jnp.minimum(start * CHUNK, N - 3200)
    delta = start * CHUNK - off
    pltpu.sync_copy(seg_hbm.at[pl.ds(off, 3200)], idx_v.at[pl.ds(0, 3200)])

    def chunk_body(c, rows_v, sem):
        dbase = delta + (c - start) * CHUNK
        _wt(rows_v, sem)

        def group_body(g, carry2):
            ids = idx_v[pl.ds(dbase + g * 16, 16)]
            mn = ids[0]
            uniform = mn == ids[15]

            @pl.when(uniform)
            def _fast():
                for j in range(NV):
                    sl = pl.ds(j * 16, 16)
                    acc = rows_v[g * 16, sl]
                    for r in range(1, 16):
                        acc = acc + rows_v[g * 16 + r, sl]
                    pool_v[mn, sl] = pool_v[mn, sl] + acc

            @pl.when(jnp.logical_not(uniform))
            def _slow():
                def row_body(r, carry3):
                    s = idx_v[pl.ds(dbase + r, 16)][0]
                    for j in range(NV):
                        sl = pl.ds(j * 16, 16)
                        pool_v[s, sl] = pool_v[s, sl] + rows_v[r, sl]
                    return carry3

                lax.fori_loop(g * 16, (g + 1) * 16, row_body, 0)

            return carry2

        lax.fori_loop(0, CHUNK // 16, group_body, 0)

        @pl.when(c + 2 < end)
        def _next():
            _ld(c + 2, rows_v, sem)

    def pair_body(p, carry):
        base = start + p * 2
        for b, (rbuf, sem) in enumerate(bufs):
            c = base + b

            @pl.when(c < end)
            def _(c=c, rbuf=rbuf, sem=sem):
                chunk_body(c, rbuf, sem)

        return carry

    lax.fori_loop(0, (cnt + 1) // 2, pair_body, 0)
    pltpu.sync_copy(pool_v, out_hbm.at[wid])


def _fc_body(part_ref, vn_ref, w_ref, b_ref, out_ref):
    pool = jnp.sum(part_ref[...], axis=0)
    x = vn_ref[...] + pool
    y = lax.dot_general(x, w_ref[...], (((1,), (1,)), ((), ())),
                        preferred_element_type=jnp.float32)
    out_ref[...] = vn_ref[...] + jnp.maximum(y + b_ref[...][None, :], 0.0)


def _fc(partials, vn_h, W, b):
    return pl.pallas_call(
        _fc_body,
        out_shape=jax.ShapeDtypeStruct((M, D), jnp.float32),
    )(partials, vn_h, W, b)


@functools.partial(
    pl.kernel,
    mesh=_mesh,
    out_type=jax.ShapeDtypeStruct((N, D), jnp.float32),
    scratch_types=[
        pltpu.VMEM((3200 + 16,), jnp.int32),
        pltpu.VMEM((BCHUNK, D), jnp.float32),
        pltpu.VMEM((BCHUNK, D), jnp.float32),
        pltpu.VMEM((BCHUNK, D), jnp.float32),
        pltpu.VMEM((BCHUNK, D), jnp.float32),
        pltpu.VMEM((M, D), jnp.float32),
        pltpu.SemaphoreType.DMA,
        pltpu.SemaphoreType.DMA,
        pltpu.SemaphoreType.DMA,
        pltpu.SemaphoreType.DMA,
    ],
)
def _bcast(h_hbm, seg_hbm, vn_hbm, out_hbm, idx_v, hin_a, hin_b,
           hout_a, hout_b, vn_v, semi_a, semi_b, semo_a, semo_b):
    wid = _worker_id()
    bufs = ((hin_a, hout_a, semi_a, semo_a), (hin_b, hout_b, semi_b, semo_b))

    def _ld(c, hin, semi):
        pltpu.async_copy(h_hbm.at[pl.ds(c * BCHUNK, BCHUNK)], hin, semi)

    def _wt_ld(hin, semi):
        pltpu.make_async_copy(h_hbm.at[pl.ds(0, BCHUNK)], hin, semi).wait()

    def _st(c, hout, semo):
        pltpu.async_copy(hout, out_hbm.at[pl.ds(c * BCHUNK, BCHUNK)], semo)

    def _wt_st(hout, semo):
        pltpu.make_async_copy(hout, out_hbm.at[pl.ds(0, BCHUNK)], semo).wait()

    pltpu.sync_copy(vn_hbm, vn_v)
    start, cnt = _chunk_range(wid, NBCHUNKS)
    end = start + cnt

    @pl.when(cnt > 0)
    def _p0():
        _ld(start, hin_a, semi_a)

    @pl.when(cnt > 1)
    def _p1():
        _ld(start + 1, hin_b, semi_b)

    off = jnp.minimum(start * BCHUNK, N - 3200)
    delta = start * BCHUNK - off
    pltpu.sync_copy(seg_hbm.at[pl.ds(off, 3200)], idx_v.at[pl.ds(0, 3200)])

    def chunk_body(c, hin, hout, semi, semo):
        dbase = delta + (c - start) * BCHUNK
        _wt_ld(hin, semi)

        @pl.when(c - 2 >= start)
        def _drain():
            _wt_st(hout, semo)

        def group_body(g, carry2):
            ids = idx_v[pl.ds(dbase + g * 16, 16)]
            mn = ids[0]
            uniform = mn == ids[15]

            @pl.when(uniform)
            def _fast():
                for j in range(NV):
                    sl = pl.ds(j * 16, 16)
                    vnr = vn_v[mn, sl]
                    for r in range(16):
                        hout[g * 16 + r, sl] = hin[g * 16 + r, sl] + vnr

            @pl.when(jnp.logical_not(uniform))
            def _slow():
                def row_body(r, carry3):
                    s = idx_v[pl.ds(dbase + r, 16)][0]
                    for j in range(NV):
                        sl = pl.ds(j * 16, 16)
                        hout[r, sl] = hin[r, sl] + vn_v[s, sl]
                    return carry3

                lax.fori_loop(g * 16, (g + 1) * 16, row_body, 0)

            return carry2

        lax.fori_loop(0, BCHUNK // 16, group_body, 0)
        _st(c, hout, semo)

        @pl.when(c + 2 < end)
        def _next():
            _ld(c + 2, hin, semi)

    def pair_body(p, carry):
        base = start + p * 2
        for b, (hin, hout, semi, semo) in enumerate(bufs):
            c = base + b

            @pl.when(c < end)
            def _(c=c, hin=hin, hout=hout, semi=semi, semo=semo):
                chunk_body(c, hin, hout, semi, semo)

        return carry

    lax.fori_loop(0, (cnt + 1) // 2, pair_body, 0)
    for b, (hin, hout, semi, semo) in enumerate(bufs):
        @pl.when(cnt > b)
        def _(hout=hout, semo=semo):
            _wt_st(hout, semo)


def kernel(h, vn_h, segment_ids, W, b):
    seg = segment_ids.astype(jnp.int32)
    partials = _segsum(h, seg)
    vn_new = _fc(partials, vn_h, W, b)
    h_new = _bcast(h, seg, vn_new)
    return (h_new, vn_new)
